# baseline (speedup 1.0000x reference)
import functools

import numpy as np

import jax
import jax.numpy as jnp
from jax import lax
from jax.experimental import pallas as pl
from jax.experimental.pallas import tpu as pltpu
from jax.experimental.pallas import tpu_sc as plsc

N_NODES = 10000
D = 128
N_PAD = 10240
E_PAD = 327680
CHUNK = 256
GJ = CHUNK // 128
ROWS_PT = N_PAD // 16
ACC_PT = N_NODES // 16
EPT_C = E_PAD // 32
NCH_C4 = EPT_C // 1024
EPT_D = E_PAD // 16
NCH_D4 = EPT_D // 1024


def _evolve_body(iw_ref, wih_ref, whh_ref, bih_ref, bhh_ref, w_ref):
    iw = iw_ref[...]
    xw = jnp.dot(iw, wih_ref[...], preferred_element_type=jnp.float32) + bih_ref[...]
    hw = jnp.dot(iw, whh_ref[...], preferred_element_type=jnp.float32) + bhh_ref[...]
    r = jax.nn.sigmoid(xw[:, :D] + hw[:, :D])
    z = jax.nn.sigmoid(xw[:, D:2 * D] + hw[:, D:2 * D])
    n = jnp.tanh(xw[:, 2 * D:] + r * hw[:, 2 * D:])
    w_ref[...] = (1.0 - z) * n + z * iw


def _xw_body(x_ref, w_ref, wp_ref, o_ref, o16_ref):
    xv = x_ref[...]
    o_ref[...] = jnp.dot(xv, w_ref[...], preferred_element_type=jnp.float32)
    o16_ref[...] = jnp.dot(
        xv, wp_ref[...], preferred_element_type=jnp.float32
    ).astype(jnp.bfloat16)


def _post_body(p0_ref, p1_ref, xw_ref, dis_ref, lwt_ref, lb_ref, o_ref):
    dis = dis_ref[...]
    inv = dis * dis
    h = p0_ref[...] + p1_ref[...] + inv * xw_ref[...]
    h = jnp.maximum(h, 0.0)
    o_ref[...] = jnp.dot(h, lwt_ref[...], preferred_element_type=jnp.float32) + lb_ref[...]


def _sc_body(row_hbm, col_hbm, ew_hbm, xwp_hbm, part_hbm, dis_hbm,
             acc_sh, deg_sh, dis_v, rowi, coli, ewv, normv, rows_bf, fbuf,
             sem):
    c = lax.axis_index("c")
    s = lax.axis_index("s")
    zero16 = jnp.zeros((16,), jnp.float32)

    scope = jax.named_scope
    with scope("ph0_zero"):
        def zrow(i, carry):
            for j in range(8):
                fbuf[i, pl.ds(j * 16, 16)] = zero16
            return carry
        lax.fori_loop(0, 128, zrow, 0)

        def zdis(i, carry):
            dis_v[pl.ds(i * 16, 16)] = zero16
            return carry
        lax.fori_loop(0, N_PAD // 16, zdis, 0)

        for z in range(5):
            pltpu.sync_copy(fbuf,
                            acc_sh.at[pl.ds(s * ROWS_PT + z * 128, 128)])
        pltpu.sync_copy(dis_v.at[pl.ds(0, ROWS_PT)],
                        deg_sh.at[pl.ds(s * ROWS_PT, ROWS_PT)])
        plsc.subcore_barrier()

    with scope("ph1_deg"):
        def deg_chunk(k, carry):
            base = (s * NCH_D4 + k) * 8
            pltpu.make_async_copy(col_hbm.at[pl.ds(base, 8)], coli, sem).start()
            pltpu.make_async_copy(ew_hbm.at[pl.ds(base, 8)], ewv, sem).start()
            pltpu.make_async_copy(col_hbm.at[pl.ds(base, 8)], coli, sem).wait()
            pltpu.make_async_copy(ew_hbm.at[pl.ds(base, 8)], ewv, sem).wait()
            for j in range(8):
                pltpu.sync_copy(ewv.at[j], deg_sh.at[coli.at[j]], add=True)
            return carry
        lax.fori_loop(0, NCH_D4, deg_chunk, 0)
        plsc.subcore_barrier()

    with scope("ph2_dis"):
        def disb(i, carry):
            pltpu.sync_copy(deg_sh.at[pl.ds(i * 1024, 1024)], normv)
            def disb2(m, c2):
                d = normv[pl.ds(m * 16, 16)] + 1.0
                xi = lax.bitcast_convert_type(d, jnp.int32)
                yi = jnp.int32(0x5F3759DF) - lax.shift_right_arithmetic(xi, 1)
                y = lax.bitcast_convert_type(yi, jnp.float32)
                for _ in range(3):
                    y = y * (1.5 - 0.5 * d * y * y)
                dis_v[pl.ds(i * 1024 + m * 16, 16)] = y
                return c2
            lax.fori_loop(0, 64, disb2, 0)
            return carry
        lax.fori_loop(0, N_PAD // 1024, disb, 0)

        @pl.when(c == 0)
        def _():
            pltpu.sync_copy(dis_v.at[pl.ds(s * ROWS_PT, ROWS_PT)],
                            dis_hbm.at[pl.ds(s * ROWS_PT, ROWS_PT)])

    def main_chunk(k, carry):
        sub = jax.named_scope
        base = c * (E_PAD // 2 // 128) + (s * NCH_C4 + k) * 8
        with sub("e_idx"):
            pltpu.make_async_copy(row_hbm.at[pl.ds(base, 8)], rowi, sem).start()
            pltpu.make_async_copy(col_hbm.at[pl.ds(base, 8)], coli, sem).start()
            pltpu.make_async_copy(ew_hbm.at[pl.ds(base, 8)], ewv, sem).start()
            pltpu.make_async_copy(row_hbm.at[pl.ds(base, 8)], rowi, sem).wait()
            pltpu.make_async_copy(col_hbm.at[pl.ds(base, 8)], coli, sem).wait()
            pltpu.make_async_copy(ew_hbm.at[pl.ds(base, 8)], ewv, sem).wait()
        with sub("e_fire"):
            pltpu.make_async_copy(xwp_hbm.at[rowi.at[0]],
                                  rows_bf.at[pl.ds(0, 128)], sem).start()
        with sub("e_norm"):
            def normb(j, cr):
                for l in range(8):
                    ir = rowi[j, pl.ds(l * 16, 16)]
                    ic = coli[j, pl.ds(l * 16, 16)]
                    w = ewv[j, pl.ds(l * 16, 16)]
                    nr = plsc.load_gather(dis_v, [ir])
                    nc = plsc.load_gather(dis_v, [ic])
                    normv[pl.ds(j * 128 + l * 16, 16)] = nr * w * nc
                return cr
            lax.fori_loop(0, 8, normb, 0)

        def group(g, cr):
            bo = (g % 2) * 128
            nbo = ((g + 1) % 2) * 128
            gn = jnp.minimum(g + 1, 7)
            with sub("e_drain"):
                pltpu.make_async_copy(xwp_hbm.at[rowi.at[g]],
                                      rows_bf.at[pl.ds(bo, 128)], sem).wait()

            @pl.when(g < 7)
            def _():
                with sub("e_fire"):
                    pltpu.make_async_copy(xwp_hbm.at[rowi.at[gn]],
                                          rows_bf.at[pl.ds(nbo, 128)],
                                          sem).start()

            with sub("e_scale"):
                @plsc.parallel_loop(0, 8, 1, unroll=2, carry=jnp.int32(0))
                def _scale(e16, cr2):
                    nv16 = normv[pl.ds(g * 128 + e16 * 16, 16)]
                    for u in range(16):
                        e = e16 * 16 + u
                        nv = nv16[u]
                        for j in range(4):
                            pk = rows_bf[bo + e, pl.ds(j * 16, 16)]
                            a = lax.bitcast_convert_type(
                                lax.shift_left(pk, 16), jnp.float32)
                            b = lax.bitcast_convert_type(
                                pk & jnp.int32(-65536), jnp.float32)
                            fbuf[e, pl.ds(j * 32, 16)] = a * nv
                            fbuf[e, pl.ds(j * 32 + 16, 16)] = b * nv
                    return cr2
            with sub("e_scat"):
                pltpu.sync_copy(fbuf, acc_sh.at[coli.at[g]], add=True)
            return cr
        lax.fori_loop(0, 8, group, 0)
        return carry
    lax.fori_loop(0, NCH_C4, main_chunk, 0)
    plsc.subcore_barrier()

    pltpu.sync_copy(acc_sh.at[pl.ds(s * ROWS_PT, ROWS_PT)],
                    part_hbm.at[c].at[pl.ds(s * ROWS_PT, ROWS_PT)])


_sc_call = functools.partial(
    pl.kernel,
    out_type=[
        jax.ShapeDtypeStruct((2, N_PAD, D), jnp.float32),
        jax.ShapeDtypeStruct((N_PAD,), jnp.float32),
    ],
    mesh=plsc.VectorSubcoreMesh(core_axis_name="c", subcore_axis_name="s"),
    compiler_params=pltpu.CompilerParams(needs_layout_passes=False,
                                         use_tc_tiling_on_sc=False),
    scratch_types=[
        pltpu.VMEM_SHARED((N_PAD, D), jnp.float32),
        pltpu.VMEM_SHARED((N_PAD,), jnp.float32),
        pltpu.VMEM((N_PAD,), jnp.float32),
        pltpu.VMEM((8, 128), jnp.int32),
        pltpu.VMEM((8, 128), jnp.int32),
        pltpu.VMEM((8, 128), jnp.float32),
        pltpu.VMEM((1024,), jnp.float32),
        pltpu.VMEM((CHUNK, D // 2), jnp.int32),
        pltpu.VMEM((128, D), jnp.float32),
        pltpu.SemaphoreType.DMA,
    ],
)(_sc_body)


_PERM = np.empty((D,), dtype=np.int32)
for _j in range(4):
    for _m in range(16):
        _PERM[32 * _j + 2 * _m] = 32 * _j + _m
        _PERM[32 * _j + 2 * _m + 1] = 32 * _j + 16 + _m

@jax.jit
def _run(x, edge_index, edge_weight, initial_weight,
         gru_w_ih, gru_w_hh, gru_b_ih, gru_b_hh, lin_w, lin_b):
    row = edge_index[0].astype(jnp.int32)
    col = edge_index[1].astype(jnp.int32)
    ew = edge_weight.astype(jnp.float32)
    e = row.shape[0]
    pad = jnp.arange(E_PAD - e, dtype=jnp.int32) % jnp.int32(N_NODES)
    row2d = jnp.concatenate([row, pad]).reshape(E_PAD // 128, 128)
    col2d = jnp.concatenate([col, pad]).reshape(E_PAD // 128, 128)
    ew2d = jnp.zeros((E_PAD,), jnp.float32).at[:e].set(ew).reshape(E_PAD // 128, 128)
    w_evo = pl.pallas_call(
        _evolve_body,
        out_shape=jax.ShapeDtypeStruct((D, D), jnp.float32),
    )(initial_weight, gru_w_ih.T, gru_w_hh.T,
      gru_b_ih.reshape(1, 3 * D), gru_b_hh.reshape(1, 3 * D))

    blk = 5000
    nblk = N_NODES // blk
    w_perm = jnp.take(w_evo, jnp.asarray(_PERM), axis=1)
    xw, xwp16 = pl.pallas_call(
        _xw_body,
        grid=(nblk,),
        in_specs=[
            pl.BlockSpec((blk, D), lambda i: (i, 0)),
            pl.BlockSpec((D, D), lambda i: (0, 0)),
            pl.BlockSpec((D, D), lambda i: (0, 0)),
        ],
        out_specs=[
            pl.BlockSpec((blk, D), lambda i: (i, 0)),
            pl.BlockSpec((blk, D), lambda i: (i, 0)),
        ],
        out_shape=[
            jax.ShapeDtypeStruct((N_NODES, D), jnp.float32),
            jax.ShapeDtypeStruct((N_NODES, D), jnp.bfloat16),
        ],
    )(x, w_evo, w_perm)

    xwp32 = lax.bitcast_convert_type(
        xwp16.reshape(N_NODES, D // 2, 2), jnp.int32)
    part, dis = _sc_call(row2d, col2d, ew2d, xwp32)

    n_t = lin_w.shape[0]
    out = pl.pallas_call(
        _post_body,
        grid=(nblk,),
        in_specs=[
            pl.BlockSpec((blk, D), lambda i: (i, 0)),
            pl.BlockSpec((blk, D), lambda i: (i, 0)),
            pl.BlockSpec((blk, D), lambda i: (i, 0)),
            pl.BlockSpec((blk, 1), lambda i: (i, 0)),
            pl.BlockSpec((D, n_t), lambda i: (0, 0)),
            pl.BlockSpec((1, n_t), lambda i: (0, 0)),
        ],
        out_specs=pl.BlockSpec((blk, n_t), lambda i: (i, 0)),
        out_shape=jax.ShapeDtypeStruct((N_NODES, n_t), jnp.float32),
    )(part[0], part[1], xw, dis.reshape(N_PAD, 1), lin_w.T,
      lin_b.reshape(1, n_t))
    return out


def kernel(x, edge_index, edge_weight, initial_weight,
           gru_w_ih, gru_w_hh, gru_b_ih, gru_b_hh, lin_w, lin_b):
    return _run(x, edge_index, edge_weight, initial_weight,
                gru_w_ih, gru_w_hh, gru_b_ih, gru_b_hh, lin_w, lin_b)


# speedup vs baseline: 1.3506x; 1.3506x over previous
import functools

import jax
import jax.numpy as jnp
from jax import lax
from jax.experimental import pallas as pl
from jax.experimental.pallas import tpu as pltpu
from jax.experimental.pallas import tpu_sc as plsc

N_NODES = 10000
D = 128
N_PAD = 10240
E_PAD = 327680
CHUNK = 256
GJ = CHUNK // 128
ROWS_PT = N_PAD // 16
ACC_PT = N_NODES // 16
EPT_C = E_PAD // 32
NCH_C4 = EPT_C // 1024
EPT_D = E_PAD // 16
NCH_D4 = EPT_D // 1024


def _evolve_body(iw_ref, wih_ref, whh_ref, bih_ref, bhh_ref, w_ref):
    iw = iw_ref[...]
    xw = jnp.dot(iw, wih_ref[...], preferred_element_type=jnp.float32) + bih_ref[...]
    hw = jnp.dot(iw, whh_ref[...], preferred_element_type=jnp.float32) + bhh_ref[...]
    r = jax.nn.sigmoid(xw[:, :D] + hw[:, :D])
    z = jax.nn.sigmoid(xw[:, D:2 * D] + hw[:, D:2 * D])
    n = jnp.tanh(xw[:, 2 * D:] + r * hw[:, 2 * D:])
    w_ref[...] = (1.0 - z) * n + z * iw


def _xw_body(x_ref, w_ref, o_ref):
    o_ref[...] = jnp.dot(x_ref[...], w_ref[...],
                         preferred_element_type=jnp.float32)


def _post_body(p0_ref, p1_ref, xw_ref, dis_ref, lwt_ref, lb_ref, o_ref):
    dis = dis_ref[...]
    inv = dis * dis
    h = p0_ref[...] + p1_ref[...] + inv * xw_ref[...]
    h = jnp.maximum(h, 0.0)
    o_ref[...] = jnp.dot(h, lwt_ref[...], preferred_element_type=jnp.float32) + lb_ref[...]


def _sc_body(row_hbm, col_hbm, ew_hbm, xw_hbm, part_hbm, dis_hbm,
             acc_sh, deg_sh, dis_v, rowi, coli, ewv, normv, rows, sem):
    c = lax.axis_index("c")
    s = lax.axis_index("s")
    zero16 = jnp.zeros((16,), jnp.float32)

    scope = jax.named_scope
    with scope("ph0_zero"):
        def zrow(i, carry):
            for j in range(8):
                rows[i, pl.ds(j * 16, 16)] = zero16
            return carry
        lax.fori_loop(0, CHUNK, zrow, 0)

        def zdis(i, carry):
            dis_v[pl.ds(i * 16, 16)] = zero16
            return carry
        lax.fori_loop(0, N_PAD // 16, zdis, 0)

        pltpu.sync_copy(rows.at[pl.ds(0, 256)],
                        acc_sh.at[pl.ds(s * ROWS_PT, 256)])
        pltpu.sync_copy(rows.at[pl.ds(0, 256)],
                        acc_sh.at[pl.ds(s * ROWS_PT + 256, 256)])
        pltpu.sync_copy(rows.at[pl.ds(0, 128)],
                        acc_sh.at[pl.ds(s * ROWS_PT + 512, 128)])
        pltpu.sync_copy(dis_v.at[pl.ds(0, ROWS_PT)],
                        deg_sh.at[pl.ds(s * ROWS_PT, ROWS_PT)])
        plsc.subcore_barrier()

    with scope("ph1_deg"):
        def deg_chunk(k, carry):
            base = (s * NCH_D4 + k) * 8
            pltpu.make_async_copy(col_hbm.at[pl.ds(base, 8)], coli, sem).start()
            pltpu.make_async_copy(ew_hbm.at[pl.ds(base, 8)], ewv, sem).start()
            pltpu.make_async_copy(col_hbm.at[pl.ds(base, 8)], coli, sem).wait()
            pltpu.make_async_copy(ew_hbm.at[pl.ds(base, 8)], ewv, sem).wait()
            for j in range(8):
                pltpu.sync_copy(ewv.at[j], deg_sh.at[coli.at[j]], add=True)
            return carry
        lax.fori_loop(0, NCH_D4, deg_chunk, 0)
        plsc.subcore_barrier()

    with scope("ph2_dis"):
        def disb(i, carry):
            pltpu.sync_copy(deg_sh.at[pl.ds(i * 1024, 1024)], normv)
            def disb2(m, c2):
                d = normv[pl.ds(m * 16, 16)] + 1.0
                xi = lax.bitcast_convert_type(d, jnp.int32)
                yi = jnp.int32(0x5F3759DF) - lax.shift_right_arithmetic(xi, 1)
                y = lax.bitcast_convert_type(yi, jnp.float32)
                for _ in range(3):
                    y = y * (1.5 - 0.5 * d * y * y)
                dis_v[pl.ds(i * 1024 + m * 16, 16)] = y
                return c2
            lax.fori_loop(0, 64, disb2, 0)
            return carry
        lax.fori_loop(0, N_PAD // 1024, disb, 0)

        @pl.when(c == 0)
        def _():
            pltpu.sync_copy(dis_v.at[pl.ds(s * ROWS_PT, ROWS_PT)],
                            dis_hbm.at[pl.ds(s * ROWS_PT, ROWS_PT)])

    def main_chunk(k, carry):
        sub = jax.named_scope
        base = c * (E_PAD // 2 // 128) + (s * NCH_C4 + k) * 8
        with sub("e_idx"):
            pltpu.make_async_copy(row_hbm.at[pl.ds(base, 8)], rowi, sem).start()
            pltpu.make_async_copy(col_hbm.at[pl.ds(base, 8)], coli, sem).start()
            pltpu.make_async_copy(ew_hbm.at[pl.ds(base, 8)], ewv, sem).start()
            pltpu.make_async_copy(row_hbm.at[pl.ds(base, 8)], rowi, sem).wait()
            pltpu.make_async_copy(col_hbm.at[pl.ds(base, 8)], coli, sem).wait()
            pltpu.make_async_copy(ew_hbm.at[pl.ds(base, 8)], ewv, sem).wait()
        with sub("e_fire"):
            pltpu.make_async_copy(xw_hbm.at[rowi.at[0]],
                                  rows.at[pl.ds(0, 128)], sem).start()
        with sub("e_norm"):
            def normb(j, cr):
                for l in range(8):
                    ir = rowi[j, pl.ds(l * 16, 16)]
                    ic = coli[j, pl.ds(l * 16, 16)]
                    w = ewv[j, pl.ds(l * 16, 16)]
                    nr = plsc.load_gather(dis_v, [ir])
                    nc = plsc.load_gather(dis_v, [ic])
                    normv[pl.ds(j * 128 + l * 16, 16)] = nr * w * nc
                return cr
            lax.fori_loop(0, 8, normb, 0)

        def group(g, cr):
            bo = (g % 2) * 128
            nbo = ((g + 1) % 2) * 128
            gn = jnp.minimum(g + 1, 7)
            with sub("e_drain"):
                pltpu.make_async_copy(xw_hbm.at[rowi.at[g]],
                                      rows.at[pl.ds(bo, 128)], sem).wait()

            @pl.when(g < 7)
            def _():
                with sub("e_fire"):
                    pltpu.make_async_copy(xw_hbm.at[rowi.at[gn]],
                                          rows.at[pl.ds(nbo, 128)],
                                          sem).start()

            with sub("e_scale"):
                def scale(e16, cr2):
                    nv16 = normv[pl.ds(g * 128 + e16 * 16, 16)]
                    for u in range(16):
                        e = bo + e16 * 16 + u
                        nv = nv16[u]
                        for j in range(8):
                            rows[e, pl.ds(j * 16, 16)] = (
                                rows[e, pl.ds(j * 16, 16)] * nv)
                    return cr2
                lax.fori_loop(0, 8, scale, 0)
            with sub("e_scat"):
                pltpu.sync_copy(rows.at[pl.ds(bo, 128)],
                                acc_sh.at[coli.at[g]], add=True)
            return cr
        lax.fori_loop(0, 8, group, 0)
        return carry
    lax.fori_loop(0, NCH_C4, main_chunk, 0)
    plsc.subcore_barrier()

    pltpu.sync_copy(acc_sh.at[pl.ds(s * ROWS_PT, ROWS_PT)],
                    part_hbm.at[c].at[pl.ds(s * ROWS_PT, ROWS_PT)])


_sc_call = functools.partial(
    pl.kernel,
    out_type=[
        jax.ShapeDtypeStruct((2, N_PAD, D), jnp.float32),
        jax.ShapeDtypeStruct((N_PAD,), jnp.float32),
    ],
    mesh=plsc.VectorSubcoreMesh(core_axis_name="c", subcore_axis_name="s"),
    compiler_params=pltpu.CompilerParams(needs_layout_passes=False),
    scratch_types=[
        pltpu.VMEM_SHARED((N_PAD, D), jnp.float32),
        pltpu.VMEM_SHARED((N_PAD,), jnp.float32),
        pltpu.VMEM((N_PAD,), jnp.float32),
        pltpu.VMEM((8, 128), jnp.int32),
        pltpu.VMEM((8, 128), jnp.int32),
        pltpu.VMEM((8, 128), jnp.float32),
        pltpu.VMEM((1024,), jnp.float32),
        pltpu.VMEM((CHUNK, D), jnp.float32),
        pltpu.SemaphoreType.DMA,
    ],
)(_sc_body)


@jax.jit
def _run(x, edge_index, edge_weight, initial_weight,
         gru_w_ih, gru_w_hh, gru_b_ih, gru_b_hh, lin_w, lin_b):
    row = edge_index[0].astype(jnp.int32)
    col = edge_index[1].astype(jnp.int32)
    ew = edge_weight.astype(jnp.float32)
    e = row.shape[0]
    pad = jnp.arange(E_PAD - e, dtype=jnp.int32) % jnp.int32(N_NODES)
    row2d = jnp.concatenate([row, pad]).reshape(E_PAD // 128, 128)
    col2d = jnp.concatenate([col, pad]).reshape(E_PAD // 128, 128)
    ew2d = jnp.zeros((E_PAD,), jnp.float32).at[:e].set(ew).reshape(E_PAD // 128, 128)
    w_evo = pl.pallas_call(
        _evolve_body,
        out_shape=jax.ShapeDtypeStruct((D, D), jnp.float32),
    )(initial_weight, gru_w_ih.T, gru_w_hh.T,
      gru_b_ih.reshape(1, 3 * D), gru_b_hh.reshape(1, 3 * D))

    blk = 5000
    nblk = N_NODES // blk
    xw = pl.pallas_call(
        _xw_body,
        grid=(nblk,),
        in_specs=[
            pl.BlockSpec((blk, D), lambda i: (i, 0)),
            pl.BlockSpec((D, D), lambda i: (0, 0)),
        ],
        out_specs=pl.BlockSpec((blk, D), lambda i: (i, 0)),
        out_shape=jax.ShapeDtypeStruct((N_NODES, D), jnp.float32),
    )(x, w_evo)

    part, dis = _sc_call(row2d, col2d, ew2d, xw)

    n_t = lin_w.shape[0]
    out = pl.pallas_call(
        _post_body,
        grid=(nblk,),
        in_specs=[
            pl.BlockSpec((blk, D), lambda i: (i, 0)),
            pl.BlockSpec((blk, D), lambda i: (i, 0)),
            pl.BlockSpec((blk, D), lambda i: (i, 0)),
            pl.BlockSpec((blk, 1), lambda i: (i, 0)),
            pl.BlockSpec((D, n_t), lambda i: (0, 0)),
            pl.BlockSpec((1, n_t), lambda i: (0, 0)),
        ],
        out_specs=pl.BlockSpec((blk, n_t), lambda i: (i, 0)),
        out_shape=jax.ShapeDtypeStruct((N_NODES, n_t), jnp.float32),
    )(part[0], part[1], xw, dis.reshape(N_PAD, 1), lin_w.T,
      lin_b.reshape(1, n_t))
    return out


def kernel(x, edge_index, edge_weight, initial_weight,
           gru_w_ih, gru_w_hh, gru_b_ih, gru_b_hh, lin_w, lin_b):
    return _run(x, edge_index, edge_weight, initial_weight,
                gru_w_ih, gru_w_hh, gru_b_ih, gru_b_hh, lin_w, lin_b)
